# Initial kernel scaffold; baseline (speedup 1.0000x reference)
import jax
import jax.numpy as jnp
from jax.experimental import pallas as pl


def kernel(node_feature, edge_index_0, edge_weight_0, edge_index_1, edge_weight_1, cheb_w_0_0, cheb_w_0_1, cheb_b_0, mlp_w_0, mlp_b_0, cheb_w_1_0, cheb_w_1_1, cheb_b_1, mlp_w_1, mlp_b_1):
    raise NotImplementedError("write your pallas kernel here")


# speedup vs baseline: 13.6978x; 13.6978x over previous
import jax
import jax.numpy as jnp
from jax import lax
from jax.experimental import pallas as pl
from jax.experimental.pallas import tpu as pltpu
from jax.experimental.pallas import tpu_sc as plsc

B, N, E, D = 8, 10000, 320000, 128
NSUB = 16
NCORE = 2
EPT = E // NSUB
CH = 128
NCH = (EPT + CH - 1) // CH
EPTP = NCH * CH
NP2 = 10240
NR = NP2 // NSUB
NR16 = NR // 16
BPC = B // NCORE
CNT = float(B * N * D)
EPS = 1e-5
F32 = jnp.float32
I32 = jnp.int32


def _spmm_body(h_hbm, rc_hbm, ew_hbm, t1_out, s_out, norm_out, parts_out,
               rcst, ewst, normst, dacc, dinvb, red, red2, gbuf,
               sdinv, acc, sem):
    cid = lax.axis_index("c")
    sid = lax.axis_index("s")
    zeros16 = jnp.zeros((16,), F32)

    def zero_vec(ref, nchunks):
        def zb(i, _):
            ref[pl.ds(i * 16, 16)] = zeros16
            return 0
        lax.fori_loop(0, nchunks, zb, 0)

    zero_vec(dacc, NP2 // 16)

    def deg_loop(j, _):
        pltpu.sync_copy(rc_hbm.at[sid, j], rcst)
        pltpu.sync_copy(ew_hbm.at[sid, j], ewst)
        for k in range(8):
            r16 = rcst[0, pl.ds(k * 16, 16)]
            w16 = ewst[0, pl.ds(k * 16, 16)]
            plsc.addupdate_scatter(dacc, [r16], w16)
        return 0
    lax.fori_loop(0, NCH, deg_loop, 0)

    pltpu.sync_copy(dacc, parts_out.at[0, sid])
    plsc.subcore_barrier()

    base = sid * NR
    zero_vec(red, NR16)
    for p in range(NSUB):
        pltpu.sync_copy(parts_out.at[0, p, pl.ds(base, NR)], red2)

        def add_red(i, _):
            red[pl.ds(i * 16, 16)] = (red[pl.ds(i * 16, 16)]
                                      + red2[pl.ds(i * 16, 16)])
            return 0
        lax.fori_loop(0, NR16, add_red, 0)

    def newton(i, _):
        d = red[pl.ds(i * 16, 16)]
        pos = d > 0.0
        safe = jnp.where(pos, d, jnp.ones((16,), F32))
        bits = plsc.bitcast(safe, I32)
        y = plsc.bitcast(jnp.full((16,), 0x5F3759DF, I32)
                         - lax.shift_right_logical(bits, 1), F32)
        nhalf = -0.5 * safe
        for _ in range(4):
            y = y * (1.5 + nhalf * y * y)
        red[pl.ds(i * 16, 16)] = jnp.where(pos, y, zeros16)
        return 0
    lax.fori_loop(0, NR16, newton, 0)

    pltpu.sync_copy(red, sdinv.at[pl.ds(base, NR)])
    plsc.subcore_barrier()
    pltpu.sync_copy(sdinv, dinvb)

    zero_vec(dacc, NP2 // 16)

    def norm_loop(j, _):
        pltpu.sync_copy(rc_hbm.at[sid, j], rcst)
        pltpu.sync_copy(ew_hbm.at[sid, j], ewst)
        for k in range(8):
            r16 = rcst[0, pl.ds(k * 16, 16)]
            c16 = rcst[1, pl.ds(k * 16, 16)]
            w16 = ewst[0, pl.ds(k * 16, 16)]
            dr = plsc.load_gather(dinvb, [r16])
            dc = plsc.load_gather(dinvb, [c16])
            nm = -(w16 * dr * dc)
            normst[0, pl.ds(k * 16, 16)] = nm
            plsc.addupdate_scatter(dacc, [r16], nm)
        pltpu.sync_copy(normst.at[0], norm_out.at[sid, j])
        return 0
    lax.fori_loop(0, NCH, norm_loop, 0)

    pltpu.sync_copy(dacc, parts_out.at[1, sid])
    plsc.subcore_barrier()

    zero_vec(red, NR16)
    for p in range(NSUB):
        pltpu.sync_copy(parts_out.at[1, p, pl.ds(base, NR)], red2)

        def add_red2(i, _):
            red[pl.ds(i * 16, 16)] = (red[pl.ds(i * 16, 16)]
                                      + red2[pl.ds(i * 16, 16)])
            return 0
        lax.fori_loop(0, NR16, add_red2, 0)

    @pl.when(cid == 0)
    def _():
        pltpu.sync_copy(red, s_out.at[pl.ds(base, NR)])

    for i in range(BPC):
        bb = cid * BPC + i

        def zg(j, _):
            for k in range(8):
                gbuf[j, pl.ds(k * 16, 16)] = zeros16
            return 0
        lax.fori_loop(0, CH, zg, 0)
        for m in range(NR // 128):
            pltpu.sync_copy(gbuf, acc.at[pl.ds(base + m * 128, 128)])
        plsc.subcore_barrier()

        boff = bb * N

        def chunk_loop(j, _):
            pltpu.sync_copy(rc_hbm.at[sid, j], rcst)
            pltpu.sync_copy(norm_out.at[sid, j], normst.at[0])
            for k in range(8):
                rcst[1, pl.ds(k * 16, 16)] = (
                    rcst[1, pl.ds(k * 16, 16)] + jnp.full((16,), boff, I32))
            pltpu.async_copy(h_hbm.at[rcst.at[1]], gbuf, sem).wait()

            def scale_group(g, _2):
                nv = normst[0, pl.ds(g * 16, 16)]
                for q in range(16):
                    r = g * 16 + q
                    bc = jnp.full((16,), nv[q], F32)
                    for k in range(8):
                        gbuf[r, pl.ds(k * 16, 16)] = (
                            gbuf[r, pl.ds(k * 16, 16)] * bc)
                return 0
            lax.fori_loop(0, CH // 16, scale_group, 0)
            pltpu.sync_copy(gbuf, acc.at[rcst.at[0]], add=True)
            return 0
        lax.fori_loop(0, NCH, chunk_loop, 0)
        plsc.subcore_barrier()

        pltpu.sync_copy(acc.at[pl.ds(base, NR)],
                        t1_out.at[bb, pl.ds(base, NR)])


def _make_spmm():
    mesh = plsc.VectorSubcoreMesh(core_axis_name="c", subcore_axis_name="s",
                                  num_cores=NCORE)
    return pl.kernel(
        _spmm_body,
        out_type=[jax.ShapeDtypeStruct((B, NP2, D), F32),
                  jax.ShapeDtypeStruct((NP2,), F32),
                  jax.ShapeDtypeStruct((NSUB, NCH, CH), F32),
                  jax.ShapeDtypeStruct((2, NSUB, NP2), F32)],
        mesh=mesh,
        compiler_params=pltpu.CompilerParams(needs_layout_passes=False),
        scratch_types=[
            pltpu.VMEM((2, CH), I32),
            pltpu.VMEM((1, CH), F32),
            pltpu.VMEM((1, CH), F32),
            pltpu.VMEM((NP2,), F32),
            pltpu.VMEM((NP2,), F32),
            pltpu.VMEM((NR,), F32),
            pltpu.VMEM((NR,), F32),
            pltpu.VMEM((CH, D), F32),
            pltpu.VMEM_SHARED((NP2,), F32),
            pltpu.VMEM_SHARED((NP2, D), F32),
            pltpu.SemaphoreType.DMA,
        ],
    )


RB = 1000


def _stats_body(x_ref, s_ref, q_ref):
    i = pl.program_id(0)

    @pl.when(i == 0)
    def _():
        s_ref[0, 0] = 0.0
        q_ref[0, 0] = 0.0

    x = x_ref[...]
    s_ref[0, 0] += jnp.sum(x)
    q_ref[0, 0] += jnp.sum(x * x)


def _stats_call(x):
    return pl.pallas_call(
        _stats_body,
        grid=(B * N // RB,),
        in_specs=[pl.BlockSpec((RB, D), lambda i: (i, 0))],
        out_specs=[pl.BlockSpec(memory_space=pltpu.SMEM),
                   pl.BlockSpec(memory_space=pltpu.SMEM)],
        out_shape=[jax.ShapeDtypeStruct((1, 1), F32),
                   jax.ShapeDtypeStruct((1, 1), F32)],
    )(x)


def _affine(s_ref, q_ref):
    mean = s_ref[0, 0] / CNT
    var = q_ref[0, 0] / CNT - mean * mean
    a = lax.rsqrt(var + EPS)
    c = -mean * a
    return a, c


def _fused1_body(h_ref, t_ref, sarr_ref, s_ref, q_ref, w0_ref, w1_ref, b_ref,
                 o_ref):
    a, c = _affine(s_ref, q_ref)
    h = h_ref[...]
    t = t_ref[0]
    w0 = w0_ref[...]
    w1 = w1_ref[...]
    cs0 = jnp.sum(w0, axis=0, keepdims=True)
    cs1 = jnp.sum(w1, axis=0, keepdims=True)
    s_col = sarr_ref[0, 0, :].reshape(RB, 1)
    acc = jnp.dot(h, w0, preferred_element_type=F32)
    acc += jnp.dot(t, w1, preferred_element_type=F32)
    o_ref[...] = (a * acc + c * cs0 + (c * s_col) * cs1
                  + b_ref[...] + h)


def _fused1_call(h, t1, sarr, ssum, ssq, w0, w1, bvec):
    return pl.pallas_call(
        _fused1_body,
        grid=(B * N // RB,),
        in_specs=[
            pl.BlockSpec((RB, D), lambda i: (i, 0)),
            pl.BlockSpec((1, RB, D),
                         lambda i: (lax.div(i, N // RB),
                                    lax.rem(i, N // RB), 0)),
            pl.BlockSpec((1, 1, RB), lambda i: (lax.rem(i, N // RB), 0, 0)),
            pl.BlockSpec(memory_space=pltpu.SMEM),
            pl.BlockSpec(memory_space=pltpu.SMEM),
            pl.BlockSpec((D, D), lambda i: (0, 0)),
            pl.BlockSpec((D, D), lambda i: (0, 0)),
            pl.BlockSpec((1, D), lambda i: (0, 0)),
        ],
        out_specs=pl.BlockSpec((RB, D), lambda i: (i, 0)),
        out_shape=jax.ShapeDtypeStruct((B * N, D), F32),
    )(h, t1, sarr, ssum, ssq, w0, w1, bvec)


def _gelu_exact(z):
    return 0.5 * z * (1.0 + lax.erf(z * (2.0 ** -0.5)))


def _fused2_body(h_ref, s_ref, q_ref, mw_ref, mb_ref, o_ref):
    a, c = _affine(s_ref, q_ref)
    h = h_ref[...]
    mw = mw_ref[...]
    csm = jnp.sum(mw, axis=0, keepdims=True)
    z = a * jnp.dot(h, mw, preferred_element_type=F32) + c * csm + mb_ref[...]
    o_ref[...] = _gelu_exact(z) + h


def _fused2_call(h, ssum, ssq, mw, mbvec):
    return pl.pallas_call(
        _fused2_body,
        grid=(B * N // RB,),
        in_specs=[
            pl.BlockSpec((RB, D), lambda i: (i, 0)),
            pl.BlockSpec(memory_space=pltpu.SMEM),
            pl.BlockSpec(memory_space=pltpu.SMEM),
            pl.BlockSpec((D, D), lambda i: (0, 0)),
            pl.BlockSpec((1, D), lambda i: (0, 0)),
        ],
        out_specs=pl.BlockSpec((RB, D), lambda i: (i, 0)),
        out_shape=jax.ShapeDtypeStruct((B * N, D), F32),
    )(h, ssum, ssq, mw, mbvec)


def _fused2f_body(h_ref, s_ref, q_ref, mw_ref, mb_ref, o_ref):
    a, c = _affine(s_ref, q_ref)
    i = pl.program_id(0)
    h = h_ref[...]
    mw = mw_ref[...]
    csm = jnp.sum(mw, axis=0, keepdims=True)
    z = a * jnp.dot(h, mw, preferred_element_type=F32) + c * csm + mb_ref[...]
    h2 = _gelu_exact(z) + h

    @pl.when(lax.rem(i, N // RB) == 0)
    def _():
        o_ref[...] = jnp.zeros((1, 1, D), F32)

    o_ref[...] += jnp.sum(h2, axis=0).reshape(1, 1, D) * (1.0 / N)


def _fused2f_call(h, ssum, ssq, mw, mbvec):
    return pl.pallas_call(
        _fused2f_body,
        grid=(B * N // RB,),
        in_specs=[
            pl.BlockSpec((RB, D), lambda i: (i, 0)),
            pl.BlockSpec(memory_space=pltpu.SMEM),
            pl.BlockSpec(memory_space=pltpu.SMEM),
            pl.BlockSpec((D, D), lambda i: (0, 0)),
            pl.BlockSpec((1, D), lambda i: (0, 0)),
        ],
        out_specs=pl.BlockSpec((1, 1, D),
                               lambda i: (lax.div(i, N // RB), 0, 0)),
        out_shape=jax.ShapeDtypeStruct((B, 1, D), F32),
    )(h, ssum, ssq, mw, mbvec).reshape(B, D)


def _prep_edges(edge_index, edge_weight):
    rc = edge_index.astype(I32).reshape(2, NSUB, EPT)
    ew = edge_weight.astype(F32).reshape(NSUB, EPT)
    pad3 = ((0, 0), (0, 0), (0, EPTP - EPT))
    pad2 = ((0, 0), (0, EPTP - EPT))
    rcp = (jnp.pad(rc, pad3).reshape(2, NSUB, NCH, CH)
           .transpose(1, 2, 0, 3))
    ewp = jnp.pad(ew, pad2).reshape(NSUB, NCH, 1, CH)
    return rcp, ewp


def kernel(node_feature, edge_index_0, edge_weight_0, edge_index_1,
           edge_weight_1, cheb_w_0_0, cheb_w_0_1, cheb_b_0, mlp_w_0, mlp_b_0,
           cheb_w_1_0, cheb_w_1_1, cheb_b_1, mlp_w_1, mlp_b_1):
    spmm = _make_spmm()
    h = node_feature.astype(F32).reshape(B * N, D)
    layers = [
        (edge_index_0, edge_weight_0, cheb_w_0_0, cheb_w_0_1, cheb_b_0,
         mlp_w_0, mlp_b_0),
        (edge_index_1, edge_weight_1, cheb_w_1_0, cheb_w_1_1, cheb_b_1,
         mlp_w_1, mlp_b_1),
    ]
    out = None
    for l, (ei, ewt, w0, w1, cb, mw, mb) in enumerate(layers):
        rcp, ewp = _prep_edges(ei, ewt)
        ssum, ssq = _stats_call(h)
        t1, s, _, _ = spmm(h, rcp, ewp)
        sarr = s[:N].reshape(N // RB, 1, RB)
        h = _fused1_call(h, t1, sarr, ssum, ssq, w0.astype(F32),
                         w1.astype(F32), cb.astype(F32).reshape(1, D))
        ssum2, ssq2 = _stats_call(h)
        if l == 0:
            h = _fused2_call(h, ssum2, ssq2, mw.astype(F32),
                             mb.astype(F32).reshape(1, D))
        else:
            out = _fused2f_call(h, ssum2, ssq2, mw.astype(F32),
                                mb.astype(F32).reshape(1, D))
    return out


# speedup vs baseline: 17.1379x; 1.2511x over previous
import jax
import jax.numpy as jnp
from jax import lax
from jax.experimental import pallas as pl
from jax.experimental.pallas import tpu as pltpu
from jax.experimental.pallas import tpu_sc as plsc

B, N, E, D = 8, 10000, 320000, 128
NSUB = 16
NCORE = 2
EPT = E // NSUB
CH = 128
NCH = (EPT + CH - 1) // CH
EPTP = NCH * CH
NP2 = 10240
NR = NP2 // NSUB
NR16 = NR // 16
BPC = B // NCORE
CNT = float(B * N * D)
EPS = 1e-5
F32 = jnp.float32
I32 = jnp.int32


def _spmm_body(h_hbm, rc_hbm, ew_hbm, t1_out, s_out, norm_out, parts_out,
               rcstA, rcstB, nmA, nmB, ewst, dwork, red, red2, gbufA, gbufB,
               sdinv, acc, semA, semB):
    cid = lax.axis_index("c")
    sid = lax.axis_index("s")
    zeros16 = jnp.zeros((16,), F32)

    def zero_vec(ref, nchunks):
        def zb(i, _):
            ref[pl.ds(i * 16, 16)] = zeros16
            return 0
        lax.fori_loop(0, nchunks, zb, 0)

    zero_vec(dwork, NP2 // 16)

    def deg_loop(j, _):
        pltpu.sync_copy(rc_hbm.at[sid, j], rcstA)
        pltpu.sync_copy(ew_hbm.at[sid, j], ewst)
        for k in range(8):
            r16 = rcstA[0, pl.ds(k * 16, 16)]
            w16 = ewst[0, pl.ds(k * 16, 16)]
            plsc.addupdate_scatter(dwork, [r16], w16)
        return 0
    lax.fori_loop(0, NCH, deg_loop, 0)

    pltpu.sync_copy(dwork, parts_out.at[0, sid])
    plsc.subcore_barrier()

    base = sid * NR
    zero_vec(red, NR16)
    for p in range(NSUB):
        pltpu.sync_copy(parts_out.at[0, p, pl.ds(base, NR)], red2)

        def add_red(i, _):
            red[pl.ds(i * 16, 16)] = (red[pl.ds(i * 16, 16)]
                                      + red2[pl.ds(i * 16, 16)])
            return 0
        lax.fori_loop(0, NR16, add_red, 0)

    def newton(i, _):
        d = red[pl.ds(i * 16, 16)]
        pos = d > 0.0
        safe = jnp.where(pos, d, jnp.ones((16,), F32))
        bits = plsc.bitcast(safe, I32)
        y = plsc.bitcast(jnp.full((16,), 0x5F3759DF, I32)
                         - lax.shift_right_logical(bits, 1), F32)
        nhalf = -0.5 * safe
        for _ in range(4):
            y = y * (1.5 + nhalf * y * y)
        red[pl.ds(i * 16, 16)] = jnp.where(pos, y, zeros16)
        return 0
    lax.fori_loop(0, NR16, newton, 0)

    pltpu.sync_copy(red, sdinv.at[pl.ds(base, NR)])
    plsc.subcore_barrier()
    pltpu.sync_copy(sdinv, dwork)

    def norm_loop(j, _):
        pltpu.sync_copy(rc_hbm.at[sid, j], rcstA)
        pltpu.sync_copy(ew_hbm.at[sid, j], ewst)
        for k in range(8):
            r16 = rcstA[0, pl.ds(k * 16, 16)]
            c16 = rcstA[1, pl.ds(k * 16, 16)]
            w16 = ewst[0, pl.ds(k * 16, 16)]
            dr = plsc.load_gather(dwork, [r16])
            dc = plsc.load_gather(dwork, [c16])
            nm = -(w16 * dr * dc)
            nmA[0, pl.ds(k * 16, 16)] = nm
        pltpu.sync_copy(nmA.at[0], norm_out.at[sid, j])
        return 0
    lax.fori_loop(0, NCH, norm_loop, 0)

    zero_vec(dwork, NP2 // 16)

    def s_loop(j, _):
        pltpu.sync_copy(rc_hbm.at[sid, j], rcstA)
        pltpu.sync_copy(norm_out.at[sid, j], nmA.at[0])
        for k in range(8):
            r16 = rcstA[0, pl.ds(k * 16, 16)]
            nm = nmA[0, pl.ds(k * 16, 16)]
            plsc.addupdate_scatter(dwork, [r16], nm)
        return 0
    lax.fori_loop(0, NCH, s_loop, 0)

    pltpu.sync_copy(dwork, parts_out.at[1, sid])
    plsc.subcore_barrier()

    zero_vec(red, NR16)
    for p in range(NSUB):
        pltpu.sync_copy(parts_out.at[1, p, pl.ds(base, NR)], red2)

        def add_red2(i, _):
            red[pl.ds(i * 16, 16)] = (red[pl.ds(i * 16, 16)]
                                      + red2[pl.ds(i * 16, 16)])
            return 0
        lax.fori_loop(0, NR16, add_red2, 0)

    @pl.when(cid == 0)
    def _():
        pltpu.sync_copy(red, s_out.at[pl.ds(base, NR)])

    for i in range(BPC):
        bb = cid * BPC + i

        def zg(j, _):
            for k in range(8):
                gbufA[j, pl.ds(k * 16, 16)] = zeros16
            return 0
        lax.fori_loop(0, CH, zg, 0)
        for m in range(NR // 128):
            pltpu.sync_copy(gbufA, acc.at[pl.ds(base + m * 128, 128)])
        plsc.subcore_barrier()

        boff = bb * N

        def edges(c, rcst, nmst):
            pltpu.sync_copy(rc_hbm.at[sid, c], rcst)
            pltpu.sync_copy(norm_out.at[sid, c], nmst.at[0])
            for k in range(8):
                rcst[1, pl.ds(k * 16, 16)] = (
                    rcst[1, pl.ds(k * 16, 16)] + jnp.full((16,), boff, I32))

        def start_g(rcst, gbuf, sem):
            pltpu.async_copy(h_hbm.at[rcst.at[1]], gbuf, sem)

        def wait_g(gbuf, sem):
            pltpu.make_async_copy(h_hbm.at[pl.ds(0, CH)], gbuf, sem).wait()

        def proc(rcst, nmst, gbuf):
            def scale_group(g, _2):
                nv = nmst[0, pl.ds(g * 16, 16)]
                for q in range(16):
                    r = g * 16 + q
                    bc = jnp.full((16,), nv[q], F32)
                    for k in range(8):
                        gbuf[r, pl.ds(k * 16, 16)] = (
                            gbuf[r, pl.ds(k * 16, 16)] * bc)
                return 0
            lax.fori_loop(0, CH // 16, scale_group, 0)
            pltpu.sync_copy(gbuf, acc.at[rcst.at[0]], add=True)

        edges(0, rcstA, nmA)
        start_g(rcstA, gbufA, semA)

        def pair_loop(p, _):
            c1 = 2 * p + 1
            edges(c1, rcstB, nmB)
            start_g(rcstB, gbufB, semB)
            wait_g(gbufA, semA)
            proc(rcstA, nmA, gbufA)
            edges(c1 + 1, rcstA, nmA)
            start_g(rcstA, gbufA, semA)
            wait_g(gbufB, semB)
            proc(rcstB, nmB, gbufB)
            return 0
        lax.fori_loop(0, (NCH - 1) // 2, pair_loop, 0)

        wait_g(gbufA, semA)
        proc(rcstA, nmA, gbufA)
        plsc.subcore_barrier()

        pltpu.sync_copy(acc.at[pl.ds(base, NR)],
                        t1_out.at[bb, pl.ds(base, NR)])


def _make_spmm():
    mesh = plsc.VectorSubcoreMesh(core_axis_name="c", subcore_axis_name="s",
                                  num_cores=NCORE)
    return pl.kernel(
        _spmm_body,
        out_type=[jax.ShapeDtypeStruct((B, NP2, D), F32),
                  jax.ShapeDtypeStruct((NP2,), F32),
                  jax.ShapeDtypeStruct((NSUB, NCH, CH), F32),
                  jax.ShapeDtypeStruct((2, NSUB, NP2), F32)],
        mesh=mesh,
        compiler_params=pltpu.CompilerParams(needs_layout_passes=False),
        scratch_types=[
            pltpu.VMEM((2, CH), I32),
            pltpu.VMEM((2, CH), I32),
            pltpu.VMEM((1, CH), F32),
            pltpu.VMEM((1, CH), F32),
            pltpu.VMEM((1, CH), F32),
            pltpu.VMEM((NP2,), F32),
            pltpu.VMEM((NR,), F32),
            pltpu.VMEM((NR,), F32),
            pltpu.VMEM((CH, D), F32),
            pltpu.VMEM((CH, D), F32),
            pltpu.VMEM_SHARED((NP2,), F32),
            pltpu.VMEM_SHARED((NP2, D), F32),
            pltpu.SemaphoreType.DMA,
            pltpu.SemaphoreType.DMA,
        ],
    )


RB = 1000


def _stats_body(x_ref, s_ref, q_ref):
    i = pl.program_id(0)

    @pl.when(i == 0)
    def _():
        s_ref[0, 0] = 0.0
        q_ref[0, 0] = 0.0

    x = x_ref[...]
    s_ref[0, 0] += jnp.sum(x)
    q_ref[0, 0] += jnp.sum(x * x)


def _stats_call(x):
    return pl.pallas_call(
        _stats_body,
        grid=(B * N // RB,),
        in_specs=[pl.BlockSpec((RB, D), lambda i: (i, 0))],
        out_specs=[pl.BlockSpec(memory_space=pltpu.SMEM),
                   pl.BlockSpec(memory_space=pltpu.SMEM)],
        out_shape=[jax.ShapeDtypeStruct((1, 1), F32),
                   jax.ShapeDtypeStruct((1, 1), F32)],
    )(x)


def _affine(s_ref, q_ref):
    mean = s_ref[0, 0] / CNT
    var = q_ref[0, 0] / CNT - mean * mean
    a = lax.rsqrt(var + EPS)
    c = -mean * a
    return a, c


def _fused1_body(h_ref, t_ref, sarr_ref, s_ref, q_ref, w0_ref, w1_ref, b_ref,
                 o_ref):
    a, c = _affine(s_ref, q_ref)
    h = h_ref[...]
    t = t_ref[0]
    w0 = w0_ref[...]
    w1 = w1_ref[...]
    cs0 = jnp.sum(w0, axis=0, keepdims=True)
    cs1 = jnp.sum(w1, axis=0, keepdims=True)
    s_col = sarr_ref[0, 0, :].reshape(RB, 1)
    acc = jnp.dot(h, w0, preferred_element_type=F32)
    acc += jnp.dot(t, w1, preferred_element_type=F32)
    o_ref[...] = (a * acc + c * cs0 + (c * s_col) * cs1
                  + b_ref[...] + h)


def _fused1_call(h, t1, sarr, ssum, ssq, w0, w1, bvec):
    return pl.pallas_call(
        _fused1_body,
        grid=(B * N // RB,),
        in_specs=[
            pl.BlockSpec((RB, D), lambda i: (i, 0)),
            pl.BlockSpec((1, RB, D),
                         lambda i: (lax.div(i, N // RB),
                                    lax.rem(i, N // RB), 0)),
            pl.BlockSpec((1, 1, RB), lambda i: (lax.rem(i, N // RB), 0, 0)),
            pl.BlockSpec(memory_space=pltpu.SMEM),
            pl.BlockSpec(memory_space=pltpu.SMEM),
            pl.BlockSpec((D, D), lambda i: (0, 0)),
            pl.BlockSpec((D, D), lambda i: (0, 0)),
            pl.BlockSpec((1, D), lambda i: (0, 0)),
        ],
        out_specs=pl.BlockSpec((RB, D), lambda i: (i, 0)),
        out_shape=jax.ShapeDtypeStruct((B * N, D), F32),
    )(h, t1, sarr, ssum, ssq, w0, w1, bvec)


def _gelu_exact(z):
    return 0.5 * z * (1.0 + lax.erf(z * (2.0 ** -0.5)))


def _fused2_body(h_ref, s_ref, q_ref, mw_ref, mb_ref, o_ref):
    a, c = _affine(s_ref, q_ref)
    h = h_ref[...]
    mw = mw_ref[...]
    csm = jnp.sum(mw, axis=0, keepdims=True)
    z = a * jnp.dot(h, mw, preferred_element_type=F32) + c * csm + mb_ref[...]
    o_ref[...] = _gelu_exact(z) + h


def _fused2_call(h, ssum, ssq, mw, mbvec):
    return pl.pallas_call(
        _fused2_body,
        grid=(B * N // RB,),
        in_specs=[
            pl.BlockSpec((RB, D), lambda i: (i, 0)),
            pl.BlockSpec(memory_space=pltpu.SMEM),
            pl.BlockSpec(memory_space=pltpu.SMEM),
            pl.BlockSpec((D, D), lambda i: (0, 0)),
            pl.BlockSpec((1, D), lambda i: (0, 0)),
        ],
        out_specs=pl.BlockSpec((RB, D), lambda i: (i, 0)),
        out_shape=jax.ShapeDtypeStruct((B * N, D), F32),
    )(h, ssum, ssq, mw, mbvec)


def _fused2f_body(h_ref, s_ref, q_ref, mw_ref, mb_ref, o_ref):
    a, c = _affine(s_ref, q_ref)
    i = pl.program_id(0)
    h = h_ref[...]
    mw = mw_ref[...]
    csm = jnp.sum(mw, axis=0, keepdims=True)
    z = a * jnp.dot(h, mw, preferred_element_type=F32) + c * csm + mb_ref[...]
    h2 = _gelu_exact(z) + h

    @pl.when(lax.rem(i, N // RB) == 0)
    def _():
        o_ref[...] = jnp.zeros((1, 1, D), F32)

    o_ref[...] += jnp.sum(h2, axis=0).reshape(1, 1, D) * (1.0 / N)


def _fused2f_call(h, ssum, ssq, mw, mbvec):
    return pl.pallas_call(
        _fused2f_body,
        grid=(B * N // RB,),
        in_specs=[
            pl.BlockSpec((RB, D), lambda i: (i, 0)),
            pl.BlockSpec(memory_space=pltpu.SMEM),
            pl.BlockSpec(memory_space=pltpu.SMEM),
            pl.BlockSpec((D, D), lambda i: (0, 0)),
            pl.BlockSpec((1, D), lambda i: (0, 0)),
        ],
        out_specs=pl.BlockSpec((1, 1, D),
                               lambda i: (lax.div(i, N // RB), 0, 0)),
        out_shape=jax.ShapeDtypeStruct((B, 1, D), F32),
    )(h, ssum, ssq, mw, mbvec).reshape(B, D)


def _prep_edges(edge_index, edge_weight):
    rc = edge_index.astype(I32).reshape(2, NSUB, EPT)
    ew = edge_weight.astype(F32).reshape(NSUB, EPT)
    pad3 = ((0, 0), (0, 0), (0, EPTP - EPT))
    pad2 = ((0, 0), (0, EPTP - EPT))
    rcp = (jnp.pad(rc, pad3).reshape(2, NSUB, NCH, CH)
           .transpose(1, 2, 0, 3))
    ewp = jnp.pad(ew, pad2).reshape(NSUB, NCH, 1, CH)
    return rcp, ewp


def kernel(node_feature, edge_index_0, edge_weight_0, edge_index_1,
           edge_weight_1, cheb_w_0_0, cheb_w_0_1, cheb_b_0, mlp_w_0, mlp_b_0,
           cheb_w_1_0, cheb_w_1_1, cheb_b_1, mlp_w_1, mlp_b_1):
    spmm = _make_spmm()
    h = node_feature.astype(F32).reshape(B * N, D)
    layers = [
        (edge_index_0, edge_weight_0, cheb_w_0_0, cheb_w_0_1, cheb_b_0,
         mlp_w_0, mlp_b_0),
        (edge_index_1, edge_weight_1, cheb_w_1_0, cheb_w_1_1, cheb_b_1,
         mlp_w_1, mlp_b_1),
    ]
    out = None
    for l, (ei, ewt, w0, w1, cb, mw, mb) in enumerate(layers):
        rcp, ewp = _prep_edges(ei, ewt)
        ssum, ssq = _stats_call(h)
        t1, s, _, _ = spmm(h, rcp, ewp)
        sarr = s[:N].reshape(N // RB, 1, RB)
        h = _fused1_call(h, t1, sarr, ssum, ssq, w0.astype(F32),
                         w1.astype(F32), cb.astype(F32).reshape(1, D))
        ssum2, ssq2 = _stats_call(h)
        if l == 0:
            h = _fused2_call(h, ssum2, ssq2, mw.astype(F32),
                             mb.astype(F32).reshape(1, D))
        else:
            out = _fused2f_call(h, ssum2, ssq2, mw.astype(F32),
                                mb.astype(F32).reshape(1, D))
    return out


# speedup vs baseline: 22.0222x; 1.2850x over previous
import jax
import jax.numpy as jnp
from jax import lax
from jax.experimental import pallas as pl
from jax.experimental.pallas import tpu as pltpu
from jax.experimental.pallas import tpu_sc as plsc

B, N, E, D = 8, 10000, 320000, 128
NSUB = 16
NCORE = 2
EPT = E // NSUB
CH = 80
NCH = (EPT + CH - 1) // CH
EPTP = NCH * CH
NP2 = 10240
NR = NP2 // NSUB
NR16 = NR // 16
BPC = B // NCORE
CNT = float(B * N * D)
EPS = 1e-5
F32 = jnp.float32
I32 = jnp.int32


def _spmm_body(h_hbm, rc_hbm, ew_hbm, t1_out, s_out, comb_out, parts_out,
               cstA, cstB, cstC, ewst, dwork, red, red2, gbufA, gbufB, gbufC,
               sdinv, acc, gsA, gsB, gsC, ssA, ssB, ssC):
    cid = lax.axis_index("c")
    sid = lax.axis_index("s")
    zeros16 = jnp.zeros((16,), F32)

    def zero_vec(ref, nchunks):
        def zb(i, _):
            ref[pl.ds(i * 16, 16)] = zeros16
            return 0
        lax.fori_loop(0, nchunks, zb, 0)

    zero_vec(dwork, NP2 // 16)

    def deg_loop(j, _):
        pltpu.sync_copy(rc_hbm.at[sid, j], cstA.at[pl.ds(0, 2)])
        pltpu.sync_copy(ew_hbm.at[sid, j], ewst)
        for k in range(CH // 16):
            r16 = cstA[0, pl.ds(k * 16, 16)]
            w16 = ewst[0, pl.ds(k * 16, 16)]
            plsc.addupdate_scatter(dwork, [r16], w16)
        return 0
    lax.fori_loop(0, NCH, deg_loop, 0)

    pltpu.sync_copy(dwork, parts_out.at[0, sid])
    plsc.subcore_barrier()

    base = sid * NR
    zero_vec(red, NR16)
    for p in range(NSUB):
        pltpu.sync_copy(parts_out.at[0, p, pl.ds(base, NR)], red2)

        def add_red(i, _):
            red[pl.ds(i * 16, 16)] = (red[pl.ds(i * 16, 16)]
                                      + red2[pl.ds(i * 16, 16)])
            return 0
        lax.fori_loop(0, NR16, add_red, 0)

    def newton(i, _):
        d = red[pl.ds(i * 16, 16)]
        pos = d > 0.0
        safe = jnp.where(pos, d, jnp.ones((16,), F32))
        bits = plsc.bitcast(safe, I32)
        y = plsc.bitcast(jnp.full((16,), 0x5F3759DF, I32)
                         - lax.shift_right_logical(bits, 1), F32)
        nhalf = -0.5 * safe
        for _ in range(4):
            y = y * (1.5 + nhalf * y * y)
        red[pl.ds(i * 16, 16)] = jnp.where(pos, y, zeros16)
        return 0
    lax.fori_loop(0, NR16, newton, 0)

    pltpu.sync_copy(red, sdinv.at[pl.ds(base, NR)])
    plsc.subcore_barrier()
    pltpu.sync_copy(sdinv, dwork)

    def norm_loop(j, _):
        pltpu.sync_copy(rc_hbm.at[sid, j], cstA.at[pl.ds(0, 2)])
        pltpu.sync_copy(ew_hbm.at[sid, j], ewst)
        for k in range(CH // 16):
            r16 = cstA[0, pl.ds(k * 16, 16)]
            c16 = cstA[1, pl.ds(k * 16, 16)]
            w16 = ewst[0, pl.ds(k * 16, 16)]
            dr = plsc.load_gather(dwork, [r16])
            dc = plsc.load_gather(dwork, [c16])
            nm = -(w16 * dr * dc)
            cstA[2, pl.ds(k * 16, 16)] = plsc.bitcast(nm, I32)
        pltpu.sync_copy(cstA, comb_out.at[sid, j])
        return 0
    lax.fori_loop(0, NCH, norm_loop, 0)

    zero_vec(dwork, NP2 // 16)

    csts = [cstA, cstB, cstC]
    gbufs = [gbufA, gbufB, gbufC]
    gsems = [gsA, gsB, gsC]
    ssems = [ssA, ssB, ssC]

    for i in range(BPC):
        bb = cid * BPC + i

        def zg(j, _):
            for k in range(8):
                gbufA[j, pl.ds(k * 16, 16)] = zeros16
            return 0
        lax.fori_loop(0, CH, zg, 0)
        for m in range(NR // CH):
            pltpu.sync_copy(gbufA, acc.at[pl.ds(base + m * CH, CH)])
        plsc.subcore_barrier()

        boff = bb * N
        do_s = (i == 0)

        def edges(c, cst):
            pltpu.sync_copy(comb_out.at[sid, c], cst)
            for k in range(CH // 16):
                cst[1, pl.ds(k * 16, 16)] = (
                    cst[1, pl.ds(k * 16, 16)] + jnp.full((16,), boff, I32))

        def start_g(cst, gbuf, sem):
            pltpu.async_copy(h_hbm.at[cst.at[1]], gbuf, sem)

        def wait_g(gbuf, sem):
            pltpu.make_async_copy(h_hbm.at[pl.ds(0, CH)], gbuf, sem).wait()

        def start_s(cst, gbuf, sem):
            pltpu.async_copy(gbuf, acc.at[cst.at[0]], sem, add=True)

        def wait_s(gbuf, sem):
            pltpu.make_async_copy(gbuf, acc.at[pl.ds(0, CH)], sem).wait()

        def proc(cst, gbuf):
            def scale_group(g, _2):
                nv = plsc.bitcast(cst[2, pl.ds(g * 16, 16)], F32)
                for q in range(16):
                    r = g * 16 + q
                    bc = jnp.full((16,), nv[q], F32)
                    for k in range(8):
                        gbuf[r, pl.ds(k * 16, 16)] = (
                            gbuf[r, pl.ds(k * 16, 16)] * bc)
                return 0
            lax.fori_loop(0, CH // 16, scale_group, 0)
            if do_s:
                def s_group(g, _2):
                    r16 = cst[0, pl.ds(g * 16, 16)]
                    nm16 = plsc.bitcast(cst[2, pl.ds(g * 16, 16)], F32)
                    plsc.addupdate_scatter(dwork, [r16], nm16)
                    return 0
                lax.fori_loop(0, CH // 16, s_group, 0)

        def body(c, s, skip_wait_s=False, issue_next=True):
            s1 = (s + 1) % 3
            if issue_next:
                if not skip_wait_s:
                    wait_s(gbufs[s1], ssems[s1])
                edges(c + 1, csts[s1])
                start_g(csts[s1], gbufs[s1], gsems[s1])
            wait_g(gbufs[s], gsems[s])
            proc(csts[s], gbufs[s])
            start_s(csts[s], gbufs[s], ssems[s])

        edges(0, csts[0])
        start_g(csts[0], gbufs[0], gsems[0])
        body(0, 0, skip_wait_s=True)
        body(1, 1, skip_wait_s=True)

        def triple_loop(t, _):
            c = 3 * t + 2
            body(c, 2)
            body(c + 1, 0)
            body(c + 2, 1)
            return 0
        TT = (NCH - 2 - 2) // 3
        lax.fori_loop(0, TT, triple_loop, 0)
        for c in range(3 * TT + 2, NCH):
            body(c, c % 3, issue_next=(c < NCH - 1))
        for c in range(NCH - 3, NCH):
            wait_s(gbufs[c % 3], ssems[c % 3])
        plsc.subcore_barrier()

        pltpu.sync_copy(acc.at[pl.ds(base, NR)],
                        t1_out.at[bb, pl.ds(base, NR)])

        if do_s:
            pltpu.sync_copy(dwork, parts_out.at[1, sid])
            plsc.subcore_barrier()
            zero_vec(red, NR16)
            for p in range(NSUB):
                pltpu.sync_copy(parts_out.at[1, p, pl.ds(base, NR)], red2)

                def add_red2(q, _):
                    red[pl.ds(q * 16, 16)] = (red[pl.ds(q * 16, 16)]
                                              + red2[pl.ds(q * 16, 16)])
                    return 0
                lax.fori_loop(0, NR16, add_red2, 0)

            @pl.when(cid == 0)
            def _():
                pltpu.sync_copy(red, s_out.at[pl.ds(base, NR)])


def _make_spmm():
    mesh = plsc.VectorSubcoreMesh(core_axis_name="c", subcore_axis_name="s",
                                  num_cores=NCORE)
    return pl.kernel(
        _spmm_body,
        out_type=[jax.ShapeDtypeStruct((B, NP2, D), F32),
                  jax.ShapeDtypeStruct((NP2,), F32),
                  jax.ShapeDtypeStruct((NSUB, NCH, 3, CH), I32),
                  jax.ShapeDtypeStruct((2, NSUB, NP2), F32)],
        mesh=mesh,
        compiler_params=pltpu.CompilerParams(needs_layout_passes=False),
        scratch_types=[
            pltpu.VMEM((3, CH), I32),
            pltpu.VMEM((3, CH), I32),
            pltpu.VMEM((3, CH), I32),
            pltpu.VMEM((1, CH), F32),
            pltpu.VMEM((NP2,), F32),
            pltpu.VMEM((NR,), F32),
            pltpu.VMEM((NR,), F32),
            pltpu.VMEM((CH, D), F32),
            pltpu.VMEM((CH, D), F32),
            pltpu.VMEM((CH, D), F32),
            pltpu.VMEM_SHARED((NP2,), F32),
            pltpu.VMEM_SHARED((NP2, D), F32),
            pltpu.SemaphoreType.DMA,
            pltpu.SemaphoreType.DMA,
            pltpu.SemaphoreType.DMA,
            pltpu.SemaphoreType.DMA,
            pltpu.SemaphoreType.DMA,
            pltpu.SemaphoreType.DMA,
        ],
    )


RB = 1000


def _stats_body(x_ref, s_ref, q_ref):
    i = pl.program_id(0)

    @pl.when(i == 0)
    def _():
        s_ref[0, 0] = 0.0
        q_ref[0, 0] = 0.0

    x = x_ref[...]
    s_ref[0, 0] += jnp.sum(x)
    q_ref[0, 0] += jnp.sum(x * x)


def _stats_call(x):
    return pl.pallas_call(
        _stats_body,
        grid=(B * N // RB,),
        in_specs=[pl.BlockSpec((RB, D), lambda i: (i, 0))],
        out_specs=[pl.BlockSpec(memory_space=pltpu.SMEM),
                   pl.BlockSpec(memory_space=pltpu.SMEM)],
        out_shape=[jax.ShapeDtypeStruct((1, 1), F32),
                   jax.ShapeDtypeStruct((1, 1), F32)],
    )(x)


def _affine(s_ref, q_ref):
    mean = s_ref[0, 0] / CNT
    var = q_ref[0, 0] / CNT - mean * mean
    a = lax.rsqrt(var + EPS)
    c = -mean * a
    return a, c


def _fused1_body(h_ref, t_ref, sarr_ref, s_ref, q_ref, w0_ref, w1_ref, b_ref,
                 o_ref):
    a, c = _affine(s_ref, q_ref)
    h = h_ref[...]
    t = t_ref[0]
    w0 = w0_ref[...]
    w1 = w1_ref[...]
    cs0 = jnp.sum(w0, axis=0, keepdims=True)
    cs1 = jnp.sum(w1, axis=0, keepdims=True)
    s_col = sarr_ref[0, 0, :].reshape(RB, 1)
    acc = jnp.dot(h, w0, preferred_element_type=F32)
    acc += jnp.dot(t, w1, preferred_element_type=F32)
    o_ref[...] = (a * acc + c * cs0 + (c * s_col) * cs1
                  + b_ref[...] + h)


def _fused1_call(h, t1, sarr, ssum, ssq, w0, w1, bvec):
    return pl.pallas_call(
        _fused1_body,
        grid=(B * N // RB,),
        in_specs=[
            pl.BlockSpec((RB, D), lambda i: (i, 0)),
            pl.BlockSpec((1, RB, D),
                         lambda i: (lax.div(i, N // RB),
                                    lax.rem(i, N // RB), 0)),
            pl.BlockSpec((1, 1, RB), lambda i: (lax.rem(i, N // RB), 0, 0)),
            pl.BlockSpec(memory_space=pltpu.SMEM),
            pl.BlockSpec(memory_space=pltpu.SMEM),
            pl.BlockSpec((D, D), lambda i: (0, 0)),
            pl.BlockSpec((D, D), lambda i: (0, 0)),
            pl.BlockSpec((1, D), lambda i: (0, 0)),
        ],
        out_specs=pl.BlockSpec((RB, D), lambda i: (i, 0)),
        out_shape=jax.ShapeDtypeStruct((B * N, D), F32),
    )(h, t1, sarr, ssum, ssq, w0, w1, bvec)


def _gelu_exact(z):
    return 0.5 * z * (1.0 + lax.erf(z * (2.0 ** -0.5)))


def _fused2_body(h_ref, s_ref, q_ref, mw_ref, mb_ref, o_ref):
    a, c = _affine(s_ref, q_ref)
    h = h_ref[...]
    mw = mw_ref[...]
    csm = jnp.sum(mw, axis=0, keepdims=True)
    z = a * jnp.dot(h, mw, preferred_element_type=F32) + c * csm + mb_ref[...]
    o_ref[...] = _gelu_exact(z) + h


def _fused2_call(h, ssum, ssq, mw, mbvec):
    return pl.pallas_call(
        _fused2_body,
        grid=(B * N // RB,),
        in_specs=[
            pl.BlockSpec((RB, D), lambda i: (i, 0)),
            pl.BlockSpec(memory_space=pltpu.SMEM),
            pl.BlockSpec(memory_space=pltpu.SMEM),
            pl.BlockSpec((D, D), lambda i: (0, 0)),
            pl.BlockSpec((1, D), lambda i: (0, 0)),
        ],
        out_specs=pl.BlockSpec((RB, D), lambda i: (i, 0)),
        out_shape=jax.ShapeDtypeStruct((B * N, D), F32),
    )(h, ssum, ssq, mw, mbvec)


def _fused2f_body(h_ref, s_ref, q_ref, mw_ref, mb_ref, o_ref):
    a, c = _affine(s_ref, q_ref)
    i = pl.program_id(0)
    h = h_ref[...]
    mw = mw_ref[...]
    csm = jnp.sum(mw, axis=0, keepdims=True)
    z = a * jnp.dot(h, mw, preferred_element_type=F32) + c * csm + mb_ref[...]
    h2 = _gelu_exact(z) + h

    @pl.when(lax.rem(i, N // RB) == 0)
    def _():
        o_ref[...] = jnp.zeros((1, 1, D), F32)

    o_ref[...] += jnp.sum(h2, axis=0).reshape(1, 1, D) * (1.0 / N)


def _fused2f_call(h, ssum, ssq, mw, mbvec):
    return pl.pallas_call(
        _fused2f_body,
        grid=(B * N // RB,),
        in_specs=[
            pl.BlockSpec((RB, D), lambda i: (i, 0)),
            pl.BlockSpec(memory_space=pltpu.SMEM),
            pl.BlockSpec(memory_space=pltpu.SMEM),
            pl.BlockSpec((D, D), lambda i: (0, 0)),
            pl.BlockSpec((1, D), lambda i: (0, 0)),
        ],
        out_specs=pl.BlockSpec((1, 1, D),
                               lambda i: (lax.div(i, N // RB), 0, 0)),
        out_shape=jax.ShapeDtypeStruct((B, 1, D), F32),
    )(h, ssum, ssq, mw, mbvec).reshape(B, D)


def _prep_edges(edge_index, edge_weight):
    rc = edge_index.astype(I32).reshape(2, NSUB, EPT)
    ew = edge_weight.astype(F32).reshape(NSUB, EPT)
    pad3 = ((0, 0), (0, 0), (0, EPTP - EPT))
    pad2 = ((0, 0), (0, EPTP - EPT))
    rcp = (jnp.pad(rc, pad3).reshape(2, NSUB, NCH, CH)
           .transpose(1, 2, 0, 3))
    ewp = jnp.pad(ew, pad2).reshape(NSUB, NCH, 1, CH)
    return rcp, ewp


def kernel(node_feature, edge_index_0, edge_weight_0, edge_index_1,
           edge_weight_1, cheb_w_0_0, cheb_w_0_1, cheb_b_0, mlp_w_0, mlp_b_0,
           cheb_w_1_0, cheb_w_1_1, cheb_b_1, mlp_w_1, mlp_b_1):
    spmm = _make_spmm()
    h = node_feature.astype(F32).reshape(B * N, D)
    layers = [
        (edge_index_0, edge_weight_0, cheb_w_0_0, cheb_w_0_1, cheb_b_0,
         mlp_w_0, mlp_b_0),
        (edge_index_1, edge_weight_1, cheb_w_1_0, cheb_w_1_1, cheb_b_1,
         mlp_w_1, mlp_b_1),
    ]
    out = None
    for l, (ei, ewt, w0, w1, cb, mw, mb) in enumerate(layers):
        rcp, ewp = _prep_edges(ei, ewt)
        ssum, ssq = _stats_call(h)
        t1, s, _, _ = spmm(h, rcp, ewp)
        sarr = s[:N].reshape(N // RB, 1, RB)
        h = _fused1_call(h, t1, sarr, ssum, ssq, w0.astype(F32),
                         w1.astype(F32), cb.astype(F32).reshape(1, D))
        ssum2, ssq2 = _stats_call(h)
        if l == 0:
            h = _fused2_call(h, ssum2, ssq2, mw.astype(F32),
                             mb.astype(F32).reshape(1, D))
        else:
            out = _fused2f_call(h, ssum2, ssq2, mw.astype(F32),
                                mb.astype(F32).reshape(1, D))
    return out
